# baseline (speedup 1.0000x reference)
import functools

import jax
import jax.numpy as jnp
from jax import lax
from jax.experimental import pallas as pl
from jax.experimental.pallas import tpu as pltpu
from jax.experimental.pallas import tpu_sc as plsc

ROWS, COLS = 4096, 200
VOCAB = 1000000
D = 64
SCALE = 8.0
NC, NS = 2, 16
NW = NC * NS
K = 128
NIB = ROWS // K
NCHUNK = COLS * NIB
CPW = NCHUNK // NW
NBUF = 2
L = 16

FV = 256
FCH = 999936 // FV
FPW = 124
FBUF = 2
SPAD = 261


def _fmt_body(tt_hbm, tail_hbm, out_hbm, stg_v, obuf_v, tail_v, isem, osem):
    wid = lax.axis_index("s") * NC + lax.axis_index("c")
    lane = lax.iota(jnp.int32, L)
    krow_c = [[lane + (kq * L + fb * D) for kq in range(D // L)]
              for fb in range(FBUF)]

    def start_in(fb, c, slot_ok):
        @pl.when(slot_ok & (c < FCH))
        def _():
            pltpu.async_copy(
                tt_hbm.at[:, pl.ds(c * FV, FV)],
                stg_v.at[pl.ds(fb * D, D), pl.ds(0, FV)],
                isem.at[fb],
            )

    for fb in range(FBUF):
        start_in(fb, wid * FPW + fb, fb < FPW)

    def outer(g, carry):
        for fb in range(FBUF):
            c = wid * FPW + g * FBUF + fb

            @pl.when(c < FCH)
            def _():
                pltpu.make_async_copy(
                    tt_hbm.at[:, pl.ds(0, FV)],
                    stg_v.at[pl.ds(fb * D, D), pl.ds(0, FV)],
                    isem.at[fb],
                ).wait()

                @pl.when(g > 0)
                def _():
                    pltpu.make_async_copy(
                        obuf_v.at[pl.ds(fb * FV, FV)],
                        out_hbm.at[pl.ds(0, FV)],
                        osem.at[fb],
                    ).wait()

                @plsc.parallel_loop(0, FV, unroll=4)
                def _(v):
                    v_idx = lane * 0 + v
                    row = fb * FV + v
                    for kq in range(D // L):
                        vals = plsc.load_gather(stg_v, [krow_c[fb][kq], v_idx])
                        obuf_v[row, pl.ds(kq * L, L)] = vals
                        obuf_v[row, pl.ds(D + kq * L, L)] = vals

                pltpu.async_copy(
                    obuf_v.at[pl.ds(fb * FV, FV)],
                    out_hbm.at[pl.ds(c * FV, FV)],
                    osem.at[fb],
                )

            start_in(fb, c + FBUF, g * FBUF + fb + FBUF < FPW)

        return carry

    lax.fori_loop(0, FPW // FBUF, outer, 0)

    for fb in range(FBUF):
        @pl.when(wid * FPW + fb < FCH)
        def _():
            pltpu.make_async_copy(
                obuf_v.at[pl.ds(fb * FV, FV)],
                out_hbm.at[pl.ds(0, FV)],
                osem.at[fb],
            ).wait()

    @pl.when(wid == 0)
    def _():
        pltpu.sync_copy(tail_hbm, tail_v)
        pltpu.sync_copy(tail_v, out_hbm.at[pl.ds(999936, 64)])


def _emb_body(idx_hbm, table_hbm, out_hbm, idx_v, rows_v, tbuf_v, gsem, osem):
    wid = lax.axis_index("s") * NC + lax.axis_index("c")

    pltpu.sync_copy(idx_hbm.at[wid], idx_v)

    lane = lax.iota(jnp.int32, L)
    row_c = [
        [jnp.full((L,), b * 8, jnp.int32) + (lane + kq * L) // 8
         for kq in range(D // L)]
        for b in range(NBUF)
    ]
    kl_c = [lax.rem(lane + kq * L, 8) for kq in range(D // L)]

    for b in range(NBUF):
        pltpu.async_copy(
            table_hbm.at[idx_v.at[b]], rows_v.at[pl.ds(b * K, K)], gsem.at[b]
        )

    def outer(g, carry):
        for b in range(NBUF):
            c = g * NBUF + b
            m = wid * CPW + c
            j = m // NIB
            iblk = m % NIB
            pltpu.make_async_copy(
                table_hbm.at[idx_v.at[0]], rows_v.at[pl.ds(b * K, K)],
                gsem.at[b],
            ).wait()

            @pl.when(g > 0)
            def _():
                pltpu.make_async_copy(
                    tbuf_v.at[pl.ds(b * 8, 8), :, pl.ds(0, K)],
                    out_hbm.at[0, :, 0],
                    osem.at[b],
                ).wait()

            @plsc.parallel_loop(0, K, unroll=8)
            def _(t):
                t_idx = lane * 0 + t
                for kq in range(D // L):
                    vals = rows_v[b * K + t, pl.ds(kq * L, L)] * SCALE
                    plsc.store_scatter(
                        tbuf_v, [row_c[b][kq], kl_c[kq], t_idx], vals
                    )

            pltpu.async_copy(
                tbuf_v.at[pl.ds(b * 8, 8), :, pl.ds(0, K)],
                out_hbm.at[j, :, iblk],
                osem.at[b],
            )

            cn = c + NBUF

            @pl.when(cn < CPW)
            def _():
                pltpu.async_copy(
                    table_hbm.at[idx_v.at[cn]], rows_v.at[pl.ds(b * K, K)],
                    gsem.at[b],
                )

        return carry

    lax.fori_loop(0, CPW // NBUF, outer, 0)

    for b in range(NBUF):
        pltpu.make_async_copy(
            tbuf_v.at[pl.ds(b * 8, 8), :, pl.ds(0, K)],
            out_hbm.at[0, :, 0],
            osem.at[b],
        ).wait()


@jax.jit
def _embed(idx, table_t, tail_dup):
    mesh = plsc.VectorSubcoreMesh(
        core_axis_name="c", subcore_axis_name="s", num_cores=NC, num_subcores=NS
    )
    fmt = pl.kernel(
        _fmt_body,
        out_type=jax.ShapeDtypeStruct((VOCAB, 2 * D), jnp.float32),
        mesh=mesh,
        compiler_params=pltpu.CompilerParams(
            use_tc_tiling_on_sc=True, needs_layout_passes=False
        ),
        scratch_types=[
            pltpu.VMEM((FBUF * D, SPAD), jnp.float32),
            pltpu.VMEM((FBUF * FV, 2 * D), jnp.float32),
            pltpu.VMEM((D, 2 * D), jnp.float32),
            pltpu.SemaphoreType.DMA((FBUF,)),
            pltpu.SemaphoreType.DMA((FBUF,)),
        ],
    )
    table2 = fmt(table_t, tail_dup)
    emb = pl.kernel(
        _emb_body,
        out_type=jax.ShapeDtypeStruct((COLS, D // 8, NIB, 8, K), jnp.float32),
        mesh=mesh,
        compiler_params=pltpu.CompilerParams(
            use_tc_tiling_on_sc=True, needs_layout_passes=False
        ),
        scratch_types=[
            pltpu.VMEM((CPW, K), jnp.int32),
            pltpu.VMEM((NBUF * K, 2 * D), jnp.float32),
            pltpu.VMEM((NBUF * D // 8, 8, K + 1), jnp.float32),
            pltpu.SemaphoreType.DMA((NBUF,)),
            pltpu.SemaphoreType.DMA((NBUF,)),
        ],
    )
    return emb(idx, table2)


def kernel(tokens, table):
    idx = tokens.T.reshape(NW, CPW, K)
    tail = table[999936:]
    tail_dup = jnp.concatenate([tail, tail], axis=1)
    out5 = _embed(idx, table.T, tail_dup)
    return jnp.transpose(out5, (2, 4, 0, 1, 3)).reshape(ROWS, COLS, D)


# speedup vs baseline: 2.1150x; 2.1150x over previous
import functools

import jax
import jax.numpy as jnp
from jax import lax
from jax.experimental import pallas as pl
from jax.experimental.pallas import tpu as pltpu
from jax.experimental.pallas import tpu_sc as plsc

ROWS, COLS = 4096, 200
VOCAB = 1000000
D = 64
SCALE = 8.0
NC, NS = 2, 16
NW = NC * NS
K = 128
NIB = ROWS // K
NCHUNK = COLS * NIB
CPW = NCHUNK // NW
NBUF = 4
L = 16


def _emb_body(idx_hbm, table_hbm, out_hbm, idx_v, rows_v, tbuf_v, gsem, osem):
    wid = lax.axis_index("s") * NC + lax.axis_index("c")

    pltpu.sync_copy(idx_hbm.at[wid], idx_v)

    for b in range(NBUF):
        pltpu.async_copy(
            table_hbm.at[idx_v.at[b]], rows_v.at[pl.ds(b * K, K), pl.ds(0, D)], gsem.at[b]
        )

    lane = lax.iota(jnp.int32, L)
    row_c = [
        [jnp.full((L,), b * 8, jnp.int32) + (lane + kq * L) // 8
         for kq in range(D // L)]
        for b in range(NBUF)
    ]
    kl_c = [lax.rem(lane + kq * L, 8) for kq in range(D // L)]

    def outer(g, carry):
        for b in range(NBUF):
            c = g * NBUF + b
            m = wid * CPW + c
            j = m // NIB
            iblk = m % NIB
            pltpu.make_async_copy(
                table_hbm.at[idx_v.at[0]], rows_v.at[pl.ds(b * K, K), pl.ds(0, D)],
                gsem.at[b],
            ).wait()

            @pl.when(g > 0)
            def _():
                pltpu.make_async_copy(
                    tbuf_v.at[pl.ds(b * 8, 8), :, pl.ds(0, K)],
                    out_hbm.at[0, :, 0],
                    osem.at[b],
                ).wait()

            @plsc.parallel_loop(0, K, unroll=8)
            def _(t):
                t_idx = lane * 0 + t
                for kq in range(D // L):
                    vals = rows_v[b * K + t, pl.ds(kq * L, L)] * SCALE
                    plsc.store_scatter(
                        tbuf_v, [row_c[b][kq], kl_c[kq], t_idx], vals
                    )

            pltpu.async_copy(
                tbuf_v.at[pl.ds(b * 8, 8), :, pl.ds(0, K)],
                out_hbm.at[j, :, iblk],
                osem.at[b],
            )

            cn = c + NBUF

            @pl.when(cn < CPW)
            def _():
                pltpu.async_copy(
                    table_hbm.at[idx_v.at[cn]], rows_v.at[pl.ds(b * K, K), pl.ds(0, D)],
                    gsem.at[b],
                )

        return carry

    lax.fori_loop(0, CPW // NBUF, outer, 0)

    for b in range(NBUF):
        pltpu.make_async_copy(
            tbuf_v.at[pl.ds(b * 8, 8), :, pl.ds(0, K)],
            out_hbm.at[0, :, 0],
            osem.at[b],
        ).wait()


@jax.jit
def _embed(idx, table):
    mesh = plsc.VectorSubcoreMesh(
        core_axis_name="c", subcore_axis_name="s", num_cores=NC, num_subcores=NS
    )
    fn = pl.kernel(
        _emb_body,
        out_type=jax.ShapeDtypeStruct((COLS, D // 8, NIB, 8, K), jnp.float32),
        mesh=mesh,
        compiler_params=pltpu.CompilerParams(use_tc_tiling_on_sc=False, needs_layout_passes=False),
        scratch_types=[
            pltpu.VMEM((CPW, K), jnp.int32),
            pltpu.VMEM((NBUF * K, D), jnp.float32),
            pltpu.VMEM((NBUF * D // 8, 8, K + 1), jnp.float32),
            pltpu.SemaphoreType.DMA((NBUF,)),
            pltpu.SemaphoreType.DMA((NBUF,)),
        ],
    )
    return fn(idx, table)


def kernel(tokens, table):
    idx = tokens.T.reshape(NW, CPW, K)
    out5 = _embed(idx, table)
    return jnp.transpose(out5, (2, 4, 0, 1, 3)).reshape(ROWS, COLS, D)
